# baseline (speedup 1.0000x reference)
import jax
import jax.numpy as jnp
from jax.experimental import pallas as pl
from jax.experimental.pallas import tpu as pltpu


def _address(m_eps, na, k, beta_r, g_r, s0_r, s1_r, s2_r, gamma_r, w_prev):
    beta = jax.nn.softplus(beta_r)
    g = jax.nn.sigmoid(g_r)
    gamma = 1.0 + jax.nn.softplus(gamma_r)
    sm = jnp.maximum(jnp.maximum(s0_r, s1_r), s2_r)
    e0 = jnp.exp(s0_r - sm)
    e1 = jnp.exp(s1_r - sm)
    e2 = jnp.exp(s2_r - sm)
    es = e0 + e1 + e2
    s0, s1, s2 = e0 / es, e1 / es, e2 / es

    nb = jnp.maximum(jnp.sqrt(jnp.sum(k * k, axis=1, keepdims=True)), 1e-8)
    dot = jnp.sum(m_eps * k[:, :, None], axis=1)
    cos = dot / (na * nb)
    z = beta * cos
    zmax = jnp.max(z, axis=1, keepdims=True)
    ez = jnp.exp(z - zmax)
    w_c = ez / jnp.sum(ez, axis=1, keepdims=True)
    w_g = g * w_c + (1.0 - g) * w_prev
    n = w_g.shape[1]
    w_s = (s0 * pltpu.roll(w_g, 1, axis=1)
           + s1 * w_g
           + s2 * pltpu.roll(w_g, n - 1, axis=1))
    w = jnp.exp2(gamma * jnp.log2(w_s))
    return w / (jnp.sum(w, axis=1, keepdims=True) + 1e-16)


def _make_kernel(G, Bc, T, D, N, V, R, H):
    RV = R * V

    def ntm_kernel(x_ref, Wx, Wr, Whh, bias, rW, rb, wW, wb, oWh, oWr, ob,
                   memT, r0, h0, c0, y_ref, mem, ws, reads, h, c):
        t = pl.program_id(0)

        @pl.when(t == 0)
        def _init():
            mem[...] = jnp.broadcast_to(memT[...][None, None], (G, Bc, V, N))
            ws[...] = jnp.zeros_like(ws[...])
            reads[...] = jnp.broadcast_to(r0[...][None], (G, Bc, RV))
            h[...] = jnp.broadcast_to(h0[...][None], (G, Bc, H))
            c[...] = jnp.broadcast_to(c0[...][None], (G, Bc, H))

        for g in range(G):
            x_t = x_ref[0, g * Bc:(g + 1) * Bc, :]
            rv = reads[g]
            hv = h[g]
            cv = c[g]
            gates = (jnp.dot(x_t, Wx[...], preferred_element_type=jnp.float32)
                     + jnp.dot(rv, Wr[...], preferred_element_type=jnp.float32)
                     + jnp.dot(hv, Whh[...], preferred_element_type=jnp.float32)
                     + bias[...])
            gi = gates[:, 0:H]
            gf = gates[:, H:2 * H]
            gg = gates[:, 2 * H:3 * H]
            go = gates[:, 3 * H:4 * H]
            c_new = jax.nn.sigmoid(gf) * cv + jax.nn.sigmoid(gi) * jnp.tanh(gg)
            co = jax.nn.sigmoid(go) * jnp.tanh(c_new)
            h[g] = co
            c[g] = c_new

            new_reads = []
            for i in range(R):
                m = mem[g]
                na = jnp.maximum(
                    jnp.sqrt(jnp.sum(m * m, axis=1)), 1e-8)

                o_r = (jnp.dot(co, rW[i], preferred_element_type=jnp.float32)
                       + rb[i])
                w_r = _address(m, na, o_r[:, 0:V],
                               o_r[:, V:V + 1], o_r[:, V + 1:V + 2],
                               o_r[:, V + 2:V + 3], o_r[:, V + 3:V + 4],
                               o_r[:, V + 4:V + 5], o_r[:, V + 5:V + 6],
                               ws[g, 2 * i])
                ws[g, 2 * i] = w_r
                new_reads.append(jnp.sum(m * w_r[:, None, :], axis=2))

                o_w = (jnp.dot(co, wW[i], preferred_element_type=jnp.float32)
                       + wb[i])
                e = jax.nn.sigmoid(o_w[:, V:2 * V])
                add = o_w[:, 2 * V:3 * V]
                w_w = _address(m, na, o_w[:, 0:V],
                               o_w[:, 3 * V:3 * V + 1],
                               o_w[:, 3 * V + 1:3 * V + 2],
                               o_w[:, 3 * V + 2:3 * V + 3],
                               o_w[:, 3 * V + 3:3 * V + 4],
                               o_w[:, 3 * V + 4:3 * V + 5],
                               o_w[:, 3 * V + 5:3 * V + 6],
                               ws[g, 2 * i + 1])
                ws[g, 2 * i + 1] = w_w
                mem[g] = (m * (1.0 - w_w[:, None, :] * e[:, :, None])
                          + w_w[:, None, :] * add[:, :, None])

            rcat = jnp.concatenate(new_reads, axis=1)
            reads[g] = rcat
            y = jax.nn.sigmoid(
                jnp.dot(co, oWh[...], preferred_element_type=jnp.float32)
                + jnp.dot(rcat, oWr[...], preferred_element_type=jnp.float32)
                + ob[...])
            y_ref[0, g * Bc:(g + 1) * Bc, :] = y

    return ntm_kernel


def kernel(x, mem_bias, lstm_h_bias, lstm_c_bias, W_ih, W_hh, b_ih, b_hh,
           read_W, read_b, write_W, write_b, read_init, out_W, out_b,
           interpret=False):
    B, T, D = x.shape
    N, V = mem_bias.shape
    R = read_W.shape[0]
    H = W_hh.shape[1]
    RV = R * V
    NC = 2
    Bc = B // NC

    f32 = jnp.float32
    xT = x.transpose(1, 0, 2)
    Wx = W_ih[:, :D].T.astype(f32)
    Wr = W_ih[:, D:].T.astype(f32)
    Whh = W_hh.T.astype(f32)
    bias = (b_ih + b_hh)[None, :].astype(f32)

    KP = 128
    rWt = jnp.transpose(read_W, (0, 2, 1)).astype(f32)
    rWt = jnp.pad(rWt, ((0, 0), (0, 0), (0, KP - (V + 6))))
    rb = jnp.pad(read_b.astype(f32), ((0, 0), (0, KP - (V + 6))))[:, None, :]

    perm = (list(range(0, V)) + list(range(V + 6, 2 * V + 6))
            + list(range(2 * V + 6, 3 * V + 6)) + list(range(V, V + 6)))
    WP = 256
    wWt = jnp.transpose(write_W[:, perm, :], (0, 2, 1)).astype(f32)
    wWt = jnp.pad(wWt, ((0, 0), (0, 0), (0, WP - (3 * V + 6))))
    wb = jnp.pad(write_b[:, perm].astype(f32),
                 ((0, 0), (0, WP - (3 * V + 6))))[:, None, :]

    oWh = out_W[:, :H].T.astype(f32)
    oWr = out_W[:, H:].T.astype(f32)
    ob = out_b[None, :].astype(f32)
    memT = mem_bias.T.astype(f32)
    r0 = read_init.reshape(1, RV).astype(f32)
    h0 = lstm_h_bias.astype(f32)
    c0 = lstm_c_bias.astype(f32)

    def const(*shape):
        return pl.BlockSpec(shape, lambda *_: tuple(0 for _ in shape))

    out = pl.pallas_call(
        _make_kernel(NC, Bc, T, D, N, V, R, H),
        grid=(T,),
        in_specs=[
            pl.BlockSpec((1, B, D), lambda t: (t, 0, 0)),
            const(D, 4 * H), const(RV, 4 * H), const(H, 4 * H), const(1, 4 * H),
            const(R, H, KP), const(R, 1, KP),
            const(R, H, WP), const(R, 1, WP),
            const(H, D), const(RV, D), const(1, D),
            const(V, N), const(1, RV), const(1, H), const(1, H),
        ],
        out_specs=pl.BlockSpec((1, B, D), lambda t: (t, 0, 0)),
        out_shape=jax.ShapeDtypeStruct((T, B, D), x.dtype),
        scratch_shapes=[
            pltpu.VMEM((NC, Bc, V, N), f32),
            pltpu.VMEM((NC, 2 * R, Bc, N), f32),
            pltpu.VMEM((NC, Bc, RV), f32),
            pltpu.VMEM((NC, Bc, H), f32),
            pltpu.VMEM((NC, Bc, H), f32),
        ],
        compiler_params=pltpu.CompilerParams(
            dimension_semantics=("arbitrary",),
            vmem_limit_bytes=56 * 1024 * 1024,
        ),
        name="ntm_scan",
        interpret=interpret,
    )(xT, Wx, Wr, Whh, bias, rWt, rb, wWt, wb, oWh, oWr, ob, memT, r0, h0, c0)
    return out.transpose(1, 0, 2)


# speedup vs baseline: 1.2138x; 1.2138x over previous
import jax
import jax.numpy as jnp
from jax.experimental import pallas as pl
from jax.experimental.pallas import tpu as pltpu


def _address(m_eps, na, k, beta_r, g_r, s0_r, s1_r, s2_r, gamma_r, w_prev):
    beta = jax.nn.softplus(beta_r)
    g = jax.nn.sigmoid(g_r)
    gamma = 1.0 + jax.nn.softplus(gamma_r)
    sm = jnp.maximum(jnp.maximum(s0_r, s1_r), s2_r)
    e0 = jnp.exp(s0_r - sm)
    e1 = jnp.exp(s1_r - sm)
    e2 = jnp.exp(s2_r - sm)
    es = e0 + e1 + e2
    s0, s1, s2 = e0 / es, e1 / es, e2 / es

    nb = jnp.maximum(jnp.sqrt(jnp.sum(k * k, axis=1, keepdims=True)), 1e-8)
    dot = jnp.sum(m_eps * k[:, :, None], axis=1)
    cos = dot / (na * nb)
    z = beta * cos
    zmax = jnp.max(z, axis=1, keepdims=True)
    ez = jnp.exp(z - zmax)
    w_c = ez / jnp.sum(ez, axis=1, keepdims=True)
    w_g = g * w_c + (1.0 - g) * w_prev
    n = w_g.shape[1]
    w_s = (s0 * pltpu.roll(w_g, 1, axis=1)
           + s1 * w_g
           + s2 * pltpu.roll(w_g, n - 1, axis=1))
    w = jnp.exp2(gamma * jnp.log2(w_s))
    return w / (jnp.sum(w, axis=1, keepdims=True) + 1e-16)


def _make_kernel(Bc, T, D, N, V, R, H):
    RV = R * V

    def ntm_kernel(x_ref, Wx, Wr, Whh, bias, rW, rb, wW, wb, oWh, oWr, ob,
                   memT, r0, h0, c0, y_ref, mem, ws, reads, h, c):
        t = pl.program_id(1)

        @pl.when(t == 0)
        def _init():
            mem[...] = jnp.broadcast_to(memT[...][None, :, :], (Bc, V, N))
            ws[...] = jnp.zeros_like(ws[...])
            reads[...] = jnp.broadcast_to(r0[...], (Bc, RV))
            h[...] = jnp.broadcast_to(h0[...], (Bc, H))
            c[...] = jnp.broadcast_to(c0[...], (Bc, H))

        x_t = x_ref[0]
        rv = reads[...]
        hv = h[...]
        cv = c[...]
        gates = (jnp.dot(x_t, Wx[...], preferred_element_type=jnp.float32)
                 + jnp.dot(rv, Wr[...], preferred_element_type=jnp.float32)
                 + jnp.dot(hv, Whh[...], preferred_element_type=jnp.float32)
                 + bias[...])
        gi = gates[:, 0:H]
        gf = gates[:, H:2 * H]
        gg = gates[:, 2 * H:3 * H]
        go = gates[:, 3 * H:4 * H]
        c_new = jax.nn.sigmoid(gf) * cv + jax.nn.sigmoid(gi) * jnp.tanh(gg)
        co = jax.nn.sigmoid(go) * jnp.tanh(c_new)
        h[...] = co
        c[...] = c_new

        new_reads = []
        for i in range(R):
            m = mem[...]
            na = jnp.maximum(
                jnp.sqrt(jnp.sum(m * m, axis=1)), 1e-8)

            o_r = jnp.dot(co, rW[i], preferred_element_type=jnp.float32) + rb[i]
            w_r = _address(m, na, o_r[:, 0:V],
                           o_r[:, V:V + 1], o_r[:, V + 1:V + 2],
                           o_r[:, V + 2:V + 3], o_r[:, V + 3:V + 4],
                           o_r[:, V + 4:V + 5], o_r[:, V + 5:V + 6],
                           ws[2 * i])
            ws[2 * i] = w_r
            new_reads.append(jnp.sum(m * w_r[:, None, :], axis=2))

            o_w = jnp.dot(co, wW[i], preferred_element_type=jnp.float32) + wb[i]
            e = jax.nn.sigmoid(o_w[:, V:2 * V])
            add = o_w[:, 2 * V:3 * V]
            w_w = _address(m, na, o_w[:, 0:V],
                           o_w[:, 3 * V:3 * V + 1], o_w[:, 3 * V + 1:3 * V + 2],
                           o_w[:, 3 * V + 2:3 * V + 3], o_w[:, 3 * V + 3:3 * V + 4],
                           o_w[:, 3 * V + 4:3 * V + 5], o_w[:, 3 * V + 5:3 * V + 6],
                           ws[2 * i + 1])
            ws[2 * i + 1] = w_w
            mem[...] = (m * (1.0 - w_w[:, None, :] * e[:, :, None])
                        + w_w[:, None, :] * add[:, :, None])

        rcat = jnp.concatenate(new_reads, axis=1)
        reads[...] = rcat
        y = jax.nn.sigmoid(
            jnp.dot(co, oWh[...], preferred_element_type=jnp.float32)
            + jnp.dot(rcat, oWr[...], preferred_element_type=jnp.float32)
            + ob[...])
        y_ref[0] = y

    return ntm_kernel


def kernel(x, mem_bias, lstm_h_bias, lstm_c_bias, W_ih, W_hh, b_ih, b_hh,
           read_W, read_b, write_W, write_b, read_init, out_W, out_b,
           interpret=False):
    B, T, D = x.shape
    N, V = mem_bias.shape
    R = read_W.shape[0]
    H = W_hh.shape[1]
    RV = R * V
    NC = 2
    Bc = B // NC

    f32 = jnp.float32
    xT = x.transpose(1, 0, 2)
    Wx = W_ih[:, :D].T.astype(f32)
    Wr = W_ih[:, D:].T.astype(f32)
    Whh = W_hh.T.astype(f32)
    bias = (b_ih + b_hh)[None, :].astype(f32)

    KP = 128
    rWt = jnp.transpose(read_W, (0, 2, 1)).astype(f32)
    rWt = jnp.pad(rWt, ((0, 0), (0, 0), (0, KP - (V + 6))))
    rb = jnp.pad(read_b.astype(f32), ((0, 0), (0, KP - (V + 6))))[:, None, :]

    perm = (list(range(0, V)) + list(range(V + 6, 2 * V + 6))
            + list(range(2 * V + 6, 3 * V + 6)) + list(range(V, V + 6)))
    WP = 256
    wWt = jnp.transpose(write_W[:, perm, :], (0, 2, 1)).astype(f32)
    wWt = jnp.pad(wWt, ((0, 0), (0, 0), (0, WP - (3 * V + 6))))
    wb = jnp.pad(write_b[:, perm].astype(f32),
                 ((0, 0), (0, WP - (3 * V + 6))))[:, None, :]

    oWh = out_W[:, :H].T.astype(f32)
    oWr = out_W[:, H:].T.astype(f32)
    ob = out_b[None, :].astype(f32)
    memT = mem_bias.T.astype(f32)
    r0 = read_init.reshape(1, RV).astype(f32)
    h0 = lstm_h_bias.astype(f32)
    c0 = lstm_c_bias.astype(f32)

    def const(*shape):
        return pl.BlockSpec(shape, lambda *_: tuple(0 for _ in shape))

    out = pl.pallas_call(
        _make_kernel(Bc, T, D, N, V, R, H),
        grid=(NC, T),
        in_specs=[
            pl.BlockSpec((1, Bc, D), lambda i, t: (t, i, 0)),
            const(D, 4 * H), const(RV, 4 * H), const(H, 4 * H), const(1, 4 * H),
            const(R, H, KP), const(R, 1, KP),
            const(R, H, WP), const(R, 1, WP),
            const(H, D), const(RV, D), const(1, D),
            const(V, N), const(1, RV), const(1, H), const(1, H),
        ],
        out_specs=pl.BlockSpec((1, Bc, D), lambda i, t: (t, i, 0)),
        out_shape=jax.ShapeDtypeStruct((T, B, D), x.dtype),
        scratch_shapes=[
            pltpu.VMEM((Bc, V, N), f32),
            pltpu.VMEM((2 * R, Bc, N), f32),
            pltpu.VMEM((Bc, RV), f32),
            pltpu.VMEM((Bc, H), f32),
            pltpu.VMEM((Bc, H), f32),
        ],
        compiler_params=pltpu.CompilerParams(
            dimension_semantics=("parallel", "arbitrary"),
            vmem_limit_bytes=56 * 1024 * 1024,
        ),
        name="ntm_scan",
        interpret=interpret,
    )(xT, Wx, Wr, Whh, bias, rWt, rb, wWt, wb, oWh, oWr, ob, memT, r0, h0, c0)
    return out.transpose(1, 0, 2)
